# baseline (speedup 1.0000x reference)
import functools

import jax
import jax.numpy as jnp
import numpy as np
from jax import lax
from jax.experimental import pallas as pl
from jax.experimental.pallas import tpu as pltpu
from jax.experimental.pallas import tpu_sc as plsc

NC = 2
NS = 16
LANES = 16
EB = 80


def _make_deg_kernel(n_edges, npad):
    ec = n_edges // (NC * NS)
    nb = ec // EB
    seg = npad // NS
    kd = 25
    mesh = plsc.VectorSubcoreMesh(core_axis_name="c", subcore_axis_name="s")

    @functools.partial(
        pl.kernel,
        out_type=jax.ShapeDtypeStruct((NC * npad,), jnp.float32),
        mesh=mesh,
        scratch_types=[
            pltpu.VMEM((nb, EB), jnp.int32),
            pltpu.VMEM((nb, EB), jnp.float32),
            pltpu.VMEM((seg,), jnp.float32),
            pltpu.VMEM_SHARED((npad,), jnp.float32),
            pltpu.SemaphoreType.DMA,
        ],
        compiler_params=pltpu.CompilerParams(needs_layout_passes=False),
    )
    def k(col_hbm, ew_hbm, deg_hbm, col_c, ew_c, seg_v, deg_sh, sem):
        c = lax.axis_index("c")
        s = lax.axis_index("s")
        wid = s * NC + c
        zero16 = jnp.zeros((LANES,), jnp.float32)

        def zbody(i, _):
            seg_v[pl.ds(i * LANES, LANES)] = zero16
            return 0

        lax.fori_loop(0, seg // LANES, zbody, 0)
        pltpu.sync_copy(seg_v, deg_sh.at[pl.ds(s * seg, seg)])
        pltpu.sync_copy(col_hbm.at[wid], col_c)
        pltpu.sync_copy(ew_hbm.at[wid], ew_c)
        plsc.subcore_barrier()

        def chunk(q, _):
            def fire(i, _):
                g = q * kd + i
                pltpu.async_copy(ew_c.at[g], deg_sh.at[col_c.at[g]], sem,
                                 add=True)
                return 0

            lax.fori_loop(0, kd, fire, 0)

            def drain(i, _):
                pltpu.make_async_copy(ew_c.at[0], deg_sh.at[col_c.at[0]],
                                      sem).wait()
                return 0

            lax.fori_loop(0, kd, drain, 0)
            return 0

        lax.fori_loop(0, nb // kd, chunk, 0)
        plsc.subcore_barrier()

        pltpu.sync_copy(deg_sh.at[pl.ds(s * seg, seg)], seg_v)
        pltpu.sync_copy(seg_v, deg_hbm.at[pl.ds(c * npad + s * seg, seg)])

    return k


def _tc_dis(deg2):
    r, cc = deg2.shape
    h = r // 2

    def body(deg_ref, dis_ref):
        d = deg_ref[...]
        dis_ref[...] = lax.rsqrt(d[:h] + d[h:] + 1.0)

    return pl.pallas_call(
        body,
        in_specs=[pl.BlockSpec((r, cc), lambda: (0, 0))],
        out_specs=pl.BlockSpec((h, cc), lambda: (0, 0)),
        out_shape=jax.ShapeDtypeStruct((h, cc), jnp.float32),
    )(deg2)


def _make_wgen_kernel(n_edges, npad):
    ec = n_edges // (NC * NS)
    nb = ec // EB
    mesh = plsc.VectorSubcoreMesh(core_axis_name="c", subcore_axis_name="s")

    @functools.partial(
        pl.kernel,
        out_type=jax.ShapeDtypeStruct((NC * NS, nb, EB), jnp.float32),
        mesh=mesh,
        scratch_types=[
            pltpu.VMEM((nb, EB), jnp.int32),
            pltpu.VMEM((nb, EB), jnp.float32),
            pltpu.VMEM((nb, EB), jnp.float32),
            pltpu.VMEM((npad,), jnp.float32),
        ],
        compiler_params=pltpu.CompilerParams(needs_layout_passes=False),
    )
    def k(row_hbm, ew_hbm, dis_hbm, w_hbm, row_c, ew_c, w_c, dis_v):
        c = lax.axis_index("c")
        s = lax.axis_index("s")
        wid = s * NC + c
        pltpu.sync_copy(dis_hbm, dis_v)
        pltpu.sync_copy(row_hbm.at[wid], row_c)
        pltpu.sync_copy(ew_hbm.at[wid], ew_c)

        def body(g, _):
            for j in range(EB // LANES):
                sl = pl.ds(j * LANES, LANES)
                d16 = plsc.load_gather(dis_v, [row_c[g, sl]])
                w_c[g, sl] = ew_c[g, sl] * d16
            return 0

        lax.fori_loop(0, nb, body, 0)
        pltpu.sync_copy(w_c, w_hbm.at[wid])

    return k


def _make_agg_kernel(n_edges, npad, d):
    ec = n_edges // (NC * NS)
    nb = ec // EB
    ch = 5
    nq = nb // ch
    rpt = npad // NS
    zr = 64
    nz = rpt // zr
    dh = d // 2
    mesh = plsc.VectorSubcoreMesh(core_axis_name="c", subcore_axis_name="s")

    @functools.partial(
        pl.kernel,
        out_type=jax.ShapeDtypeStruct((NC, npad, d), jnp.float32),
        mesh=mesh,
        scratch_types=[
            pltpu.VMEM((2 * ch, EB), jnp.int32),
            pltpu.VMEM((2 * ch, EB), jnp.float32),
            pltpu.VMEM((EB, dh), jnp.int32),
            pltpu.VMEM((EB, dh), jnp.int32),
            pltpu.VMEM((EB, d), jnp.float32),
            pltpu.VMEM((EB, d), jnp.float32),
            pltpu.VMEM((EB,), jnp.int32),
            pltpu.VMEM((EB,), jnp.int32),
            pltpu.VMEM((EB,), jnp.int32),
            pltpu.VMEM((EB,), jnp.int32),
            pltpu.VMEM_SHARED((npad, d), jnp.float32),
            pltpu.SemaphoreType.DMA,
            pltpu.SemaphoreType.DMA,
            pltpu.SemaphoreType.DMA,
            pltpu.SemaphoreType.DMA,
            pltpu.SemaphoreType.DMA,
        ],
        compiler_params=pltpu.CompilerParams(
            needs_layout_passes=False, use_tc_tiling_on_sc=False),
    )
    def k(tbl_hbm, packed_hbm, w_hbm, accp_hbm,
          packed_cc, w_cc, rows_0, rows_1, msg_0, msg_1,
          row_v0, row_v1, col_v0, col_v1,
          acc, sg0, sg1, ss0, ss1, smeta):
        c = lax.axis_index("c")
        s = lax.axis_index("s")
        wid = s * NC + c
        zero16 = jnp.zeros((LANES,), jnp.float32)

        def zbody(i, _):
            for kk in range(d // LANES):
                msg_0[i, pl.ds(kk * LANES, LANES)] = zero16
            return 0

        lax.fori_loop(0, zr, zbody, 0)
        for j in range(nz):
            pltpu.sync_copy(msg_0.at[pl.ds(0, zr)],
                            acc.at[pl.ds(s * rpt + j * zr, zr)])
        pltpu.sync_copy(packed_hbm.at[wid, 0], packed_cc.at[pl.ds(0, ch)])
        pltpu.sync_copy(w_hbm.at[wid, 0], w_cc.at[pl.ds(0, ch)])
        plsc.subcore_barrier()

        def meta_wait():
            pltpu.make_async_copy(packed_hbm.at[wid, 0],
                                  packed_cc.at[pl.ds(0, ch)], smeta).wait()
            pltpu.make_async_copy(w_hbm.at[wid, 0],
                                  w_cc.at[pl.ds(0, ch)], smeta).wait()

        def unpack(g, row_v, col_v):
            @pl.when(jnp.logical_and(g % ch == 0, g > 0))
            def _():
                meta_wait()

            ridx = ((g // ch) % 2) * ch + g % ch
            for j in range(EB // LANES):
                sl = pl.ds(j * LANES, LANES)
                p16 = packed_cc[ridx, sl]
                row_v[sl] = lax.bitwise_and(p16, jnp.int32(0xFFFF))
                col_v[sl] = lax.shift_right_logical(p16, jnp.int32(16))

        def phase(g, cur, nxt):
            rows_cur, msg_cur, sg_cur, ss_cur, row_v_cur, col_v_cur = cur
            rows_nxt, msg_nxt, sg_nxt, ss_nxt, row_v_nxt, col_v_nxt = nxt

            @pl.when(jnp.logical_and(g % ch == 0, g + ch < nb))
            def _():
                qn = g // ch + 1

                @pl.when(qn % 2 == 0)
                def _():
                    pltpu.async_copy(packed_hbm.at[wid, qn],
                                     packed_cc.at[pl.ds(0, ch)], smeta)
                    pltpu.async_copy(w_hbm.at[wid, qn],
                                     w_cc.at[pl.ds(0, ch)], smeta)

                @pl.when(qn % 2 == 1)
                def _():
                    pltpu.async_copy(packed_hbm.at[wid, qn],
                                     packed_cc.at[pl.ds(ch, ch)], smeta)
                    pltpu.async_copy(w_hbm.at[wid, qn],
                                     w_cc.at[pl.ds(ch, ch)], smeta)

            @pl.when(g >= 1)
            def _():
                pltpu.make_async_copy(msg_nxt, acc.at[col_v_nxt],
                                      ss_nxt).wait()

            @pl.when(g + 1 < nb)
            def _():
                unpack(g + 1, row_v_nxt, col_v_nxt)
                pltpu.async_copy(tbl_hbm.at[row_v_nxt], rows_nxt, sg_nxt)

            pltpu.make_async_copy(tbl_hbm.at[row_v_cur], rows_cur,
                                  sg_cur).wait()

            ridx = ((g // ch) % 2) * ch + g % ch
            for j in range(EB // LANES):
                sl = pl.ds(j * LANES, LANES)
                w16 = w_cc[ridx, sl]
                for l in range(LANES):
                    wb = jnp.full((LANES,), w16[l], jnp.float32)
                    ee = j * LANES + l
                    for kk in range(dh // LANES):
                        v = rows_cur[ee, pl.ds(kk * LANES, LANES)]
                        lo = plsc.bitcast(
                            lax.shift_left(v, jnp.int32(16)), jnp.float32)
                        hi = plsc.bitcast(
                            lax.bitwise_and(v, jnp.int32(-65536)),
                            jnp.float32)
                        msg_cur[ee, pl.ds(kk * 2 * LANES, LANES)] = lo * wb
                        msg_cur[ee, pl.ds((kk * 2 + 1) * LANES, LANES)] = (
                            hi * wb)

            pltpu.async_copy(msg_cur, acc.at[col_v_cur], ss_cur, add=True)

        b0 = (rows_0, msg_0, sg0, ss0, row_v0, col_v0)
        b1 = (rows_1, msg_1, sg1, ss1, row_v1, col_v1)

        unpack(0, row_v0, col_v0)
        pltpu.async_copy(tbl_hbm.at[row_v0], rows_0, sg0)

        def body(i, _):
            g = i * 2
            phase(g, b0, b1)

            @pl.when(g + 1 < nb)
            def _():
                phase(g + 1, b1, b0)

            return 0

        lax.fori_loop(0, (nb + 1) // 2, body, 0)
        last = [b0, b1][(nb - 1) % 2]
        pltpu.make_async_copy(last[1], acc.at[last[5]], last[3]).wait()
        plsc.subcore_barrier()
        for j in range(nz):
            sl = pl.ds(s * rpt + j * zr, zr)
            pltpu.sync_copy(acc.at[sl], accp_hbm.at[c, sl])

    return k


def _tc_matmul1(x, w1, bm):
    m, cin = x.shape
    ch = w1.shape[1]
    hd = ch // 2

    def body(x_ref, w_ref, h0_ref, h1_ref):
        h = jnp.dot(x_ref[...], w_ref[...], preferred_element_type=jnp.float32)
        h0_ref[...] = h[:, :hd]
        h1_ref[...] = h[:, hd:]

    return pl.pallas_call(
        body,
        grid=(m // bm,),
        in_specs=[
            pl.BlockSpec((bm, cin), lambda i: (i, 0)),
            pl.BlockSpec((cin, ch), lambda i: (0, 0)),
        ],
        out_specs=[
            pl.BlockSpec((bm, hd), lambda i: (i, 0)),
            pl.BlockSpec((bm, hd), lambda i: (i, 0)),
        ],
        out_shape=[jax.ShapeDtypeStruct((m, hd), jnp.float32)] * 2,
    )(x, w1)


def _tc_mid(p00, p01, p10, p11, h10, h11, dis2, b10, b11, w2a, w2b, bm):
    m, hd = h10.shape
    cout = w2a.shape[1]

    def body(p00_r, p01_r, p10_r, p11_r, h10_r, h11_r, d_r, b10_r, b11_r,
             w2a_r, w2b_r, h2_ref):
        dcol = d_r[...]
        d2 = dcol * dcol
        a0 = jnp.maximum(dcol * (p00_r[...] + p01_r[...]) + d2 * h10_r[...]
                         + b10_r[...], 0.0)
        a1 = jnp.maximum(dcol * (p10_r[...] + p11_r[...]) + d2 * h11_r[...]
                         + b11_r[...], 0.0)
        h2_ref[...] = (jnp.dot(a0, w2a_r[...], preferred_element_type=jnp.float32)
                       + jnp.dot(a1, w2b_r[...], preferred_element_type=jnp.float32))

    mspec = pl.BlockSpec((bm, hd), lambda i: (i, 0))
    return pl.pallas_call(
        body,
        grid=(m // bm,),
        in_specs=[
            mspec, mspec, mspec, mspec, mspec, mspec,
            pl.BlockSpec((bm, 1), lambda i: (i, 0)),
            pl.BlockSpec((1, hd), lambda i: (0, 0)),
            pl.BlockSpec((1, hd), lambda i: (0, 0)),
            pl.BlockSpec((hd, cout), lambda i: (0, 0)),
            pl.BlockSpec((hd, cout), lambda i: (0, 0)),
        ],
        out_specs=pl.BlockSpec((bm, cout), lambda i: (i, 0)),
        out_shape=jax.ShapeDtypeStruct((m, cout), jnp.float32),
    )(p00, p01, p10, p11, h10, h11, dis2, b10, b11, w2a, w2b)


def _tc_final(q0, q1, h2, dis2, b2r, bm):
    m, cout = h2.shape

    def body(q0_r, q1_r, h2_r, d_r, b_r, out_ref):
        dcol = d_r[...]
        out_ref[...] = (dcol * (q0_r[...] + q1_r[...])
                        + dcol * dcol * h2_r[...] + b_r[...])

    mspec = pl.BlockSpec((bm, cout), lambda i: (i, 0))
    return pl.pallas_call(
        body,
        grid=(m // bm,),
        in_specs=[
            mspec, mspec, mspec,
            pl.BlockSpec((bm, 1), lambda i: (i, 0)),
            pl.BlockSpec((1, cout), lambda i: (0, 0)),
        ],
        out_specs=mspec,
        out_shape=jax.ShapeDtypeStruct((m, cout), jnp.float32),
    )(q0, q1, h2, dis2, b2r)


def kernel(x, edge_index, edge_weight, W1, b1, W2, b2):
    n, cin = x.shape
    e = edge_index.shape[1]
    ch = W1.shape[1]
    cout = W2.shape[1]
    hd = ch // 2
    npad = ((n + NS * LANES - 1) // (NS * LANES)) * (NS * LANES)
    bm = 1000
    nw = NC * NS
    nb = e // (nw * EB)

    row = edge_index[0].reshape(nw, nb, EB)
    col = edge_index[1].reshape(nw, nb, EB)
    ew = edge_weight.reshape(nw, nb, EB)
    packed = (col << 16) | row

    chq = 5
    nq = nb // chq
    packed4 = packed.reshape(nw, nq, chq, EB)

    perm = np.arange(hd)
    perm = (perm & ~31) + ((perm % 32) >> 1) + (perm % 2) * 16

    def to_tbl(h):
        hb = h[:, perm].astype(jnp.bfloat16)
        return lax.bitcast_convert_type(
            hb.reshape(n, hd // 2, 2), jnp.int32)

    h10, h11 = _tc_matmul1(x, W1, bm)
    deg_p = _make_deg_kernel(e, npad)(col, ew)
    dis_full = _tc_dis(deg_p.reshape(2 * npad // 128, 128)).reshape(npad)
    w4 = _make_wgen_kernel(e, npad)(row, ew, dis_full).reshape(nw, nq, chq, EB)
    agg_h = _make_agg_kernel(e, npad, hd)
    acc0 = agg_h(to_tbl(h10), packed4, w4)
    acc1 = agg_h(to_tbl(h11), packed4, w4)

    dis2 = dis_full[:n].reshape(n, 1)
    h2 = _tc_mid(acc0[0, :n], acc0[1, :n], acc1[0, :n], acc1[1, :n],
                 h10, h11, dis2,
                 b1[:hd].reshape(1, hd), b1[hd:].reshape(1, hd),
                 W2[:hd], W2[hd:], bm)

    acc2 = _make_agg_kernel(e, npad, cout)(to_tbl(h2), packed4, w4)
    out = _tc_final(acc2[0, :n], acc2[1, :n], h2, dis2, b2.reshape(1, cout), bm)
    return out


# speedup vs baseline: 1.0984x; 1.0984x over previous
import functools

import jax
import jax.numpy as jnp
import numpy as np
from jax import lax
from jax.experimental import pallas as pl
from jax.experimental.pallas import tpu as pltpu
from jax.experimental.pallas import tpu_sc as plsc

NC = 2
NS = 16
LANES = 16
EB = 80


def _make_deg_kernel(n_edges, npad):
    ec = n_edges // (NC * NS)
    nb = ec // EB
    seg = npad // NS
    kd = 25
    mesh = plsc.VectorSubcoreMesh(core_axis_name="c", subcore_axis_name="s")

    @functools.partial(
        pl.kernel,
        out_type=jax.ShapeDtypeStruct((NC * npad,), jnp.float32),
        mesh=mesh,
        scratch_types=[
            pltpu.VMEM((nb, EB), jnp.int32),
            pltpu.VMEM((nb, EB), jnp.float32),
            pltpu.VMEM((seg,), jnp.float32),
            pltpu.VMEM_SHARED((npad,), jnp.float32),
            pltpu.SemaphoreType.DMA,
        ],
        compiler_params=pltpu.CompilerParams(needs_layout_passes=False),
    )
    def k(col_hbm, ew_hbm, deg_hbm, col_c, ew_c, seg_v, deg_sh, sem):
        c = lax.axis_index("c")
        s = lax.axis_index("s")
        wid = s * NC + c
        zero16 = jnp.zeros((LANES,), jnp.float32)

        def zbody(i, _):
            seg_v[pl.ds(i * LANES, LANES)] = zero16
            return 0

        lax.fori_loop(0, seg // LANES, zbody, 0)
        pltpu.sync_copy(seg_v, deg_sh.at[pl.ds(s * seg, seg)])
        pltpu.sync_copy(col_hbm.at[wid], col_c)
        pltpu.sync_copy(ew_hbm.at[wid], ew_c)
        plsc.subcore_barrier()

        def chunk(q, _):
            def fire(i, _):
                g = q * kd + i
                pltpu.async_copy(ew_c.at[g], deg_sh.at[col_c.at[g]], sem,
                                 add=True)
                return 0

            lax.fori_loop(0, kd, fire, 0)

            def drain(i, _):
                pltpu.make_async_copy(ew_c.at[0], deg_sh.at[col_c.at[0]],
                                      sem).wait()
                return 0

            lax.fori_loop(0, kd, drain, 0)
            return 0

        lax.fori_loop(0, nb // kd, chunk, 0)
        plsc.subcore_barrier()

        pltpu.sync_copy(deg_sh.at[pl.ds(s * seg, seg)], seg_v)
        pltpu.sync_copy(seg_v, deg_hbm.at[pl.ds(c * npad + s * seg, seg)])

    return k


def _tc_dis(deg2):
    r, cc = deg2.shape
    h = r // 2

    def body(deg_ref, dis_ref):
        d = deg_ref[...]
        dis_ref[...] = lax.rsqrt(d[:h] + d[h:] + 1.0)

    return pl.pallas_call(
        body,
        in_specs=[pl.BlockSpec((r, cc), lambda: (0, 0))],
        out_specs=pl.BlockSpec((h, cc), lambda: (0, 0)),
        out_shape=jax.ShapeDtypeStruct((h, cc), jnp.float32),
    )(deg2)


def _make_wgen_kernel(n_edges, npad):
    ec = n_edges // (NC * NS)
    nb = ec // EB
    mesh = plsc.VectorSubcoreMesh(core_axis_name="c", subcore_axis_name="s")

    @functools.partial(
        pl.kernel,
        out_type=jax.ShapeDtypeStruct((NC * NS, nb, EB), jnp.float32),
        mesh=mesh,
        scratch_types=[
            pltpu.VMEM((nb, EB), jnp.int32),
            pltpu.VMEM((nb, EB), jnp.float32),
            pltpu.VMEM((nb, EB), jnp.float32),
            pltpu.VMEM((npad,), jnp.float32),
        ],
        compiler_params=pltpu.CompilerParams(needs_layout_passes=False),
    )
    def k(row_hbm, ew_hbm, dis_hbm, w_hbm, row_c, ew_c, w_c, dis_v):
        c = lax.axis_index("c")
        s = lax.axis_index("s")
        wid = s * NC + c
        pltpu.sync_copy(dis_hbm, dis_v)
        pltpu.sync_copy(row_hbm.at[wid], row_c)
        pltpu.sync_copy(ew_hbm.at[wid], ew_c)

        def body(g, _):
            for j in range(EB // LANES):
                sl = pl.ds(j * LANES, LANES)
                d16 = plsc.load_gather(dis_v, [row_c[g, sl]])
                w_c[g, sl] = ew_c[g, sl] * d16
            return 0

        lax.fori_loop(0, nb, body, 0)
        pltpu.sync_copy(w_c, w_hbm.at[wid])

    return k


def _make_agg_kernel(n_edges, npad, d):
    ec = n_edges // (NC * NS)
    nb = ec // EB
    ch = 25
    nq = nb // ch
    rpt = npad // NS
    zr = 64
    nz = rpt // zr
    mesh = plsc.VectorSubcoreMesh(core_axis_name="c", subcore_axis_name="s")

    @functools.partial(
        pl.kernel,
        out_type=jax.ShapeDtypeStruct((NC, npad, d), jnp.float32),
        mesh=mesh,
        scratch_types=[
            pltpu.VMEM((2 * ch, EB), jnp.int32),
            pltpu.VMEM((2 * ch, EB), jnp.float32),
            pltpu.VMEM((EB, d), jnp.float32),
            pltpu.VMEM((EB, d), jnp.float32),
            pltpu.VMEM((EB, d), jnp.float32),
            pltpu.VMEM((EB,), jnp.int32),
            pltpu.VMEM((EB,), jnp.int32),
            pltpu.VMEM((EB,), jnp.int32),
            pltpu.VMEM((EB,), jnp.int32),
            pltpu.VMEM((EB,), jnp.int32),
            pltpu.VMEM((EB,), jnp.int32),
            pltpu.VMEM_SHARED((npad, d), jnp.float32),
            pltpu.SemaphoreType.DMA,
            pltpu.SemaphoreType.DMA,
            pltpu.SemaphoreType.DMA,
            pltpu.SemaphoreType.DMA,
            pltpu.SemaphoreType.DMA,
            pltpu.SemaphoreType.DMA,
            pltpu.SemaphoreType.DMA,
        ],
        compiler_params=pltpu.CompilerParams(needs_layout_passes=False),
    )
    def k(tbl_hbm, packed_hbm, w_hbm, accp_hbm,
          packed_cc, w_cc, rows_0, rows_1, rows_2,
          row_v0, row_v1, row_v2, col_v0, col_v1, col_v2,
          acc, sg0, sg1, sg2, ss0, ss1, ss2, smeta):
        c = lax.axis_index("c")
        s = lax.axis_index("s")
        wid = s * NC + c
        zero16 = jnp.zeros((LANES,), jnp.float32)

        def zbody(i, _):
            for kk in range(d // LANES):
                rows_0[i, pl.ds(kk * LANES, LANES)] = zero16
            return 0

        lax.fori_loop(0, zr, zbody, 0)
        for j in range(nz):
            pltpu.sync_copy(rows_0.at[pl.ds(0, zr)],
                            acc.at[pl.ds(s * rpt + j * zr, zr)])
        pltpu.sync_copy(packed_hbm.at[wid, 0], packed_cc.at[pl.ds(0, ch)])
        pltpu.sync_copy(w_hbm.at[wid, 0], w_cc.at[pl.ds(0, ch)])
        plsc.subcore_barrier()

        def meta_wait():
            pltpu.make_async_copy(packed_hbm.at[wid, 0],
                                  packed_cc.at[pl.ds(0, ch)], smeta).wait()
            pltpu.make_async_copy(w_hbm.at[wid, 0],
                                  w_cc.at[pl.ds(0, ch)], smeta).wait()

        def unpack(g, row_v, col_v):
            @pl.when(jnp.logical_and(g % ch == 0, g > 0))
            def _():
                meta_wait()

            ridx = ((g // ch) % 2) * ch + g % ch
            for j in range(EB // LANES):
                sl = pl.ds(j * LANES, LANES)
                p16 = packed_cc[ridx, sl]
                row_v[sl] = lax.bitwise_and(p16, jnp.int32(0xFFFF))
                col_v[sl] = lax.shift_right_logical(p16, jnp.int32(16))

        def phase(g, cur, nxt):
            rows_cur, sg_cur, ss_cur, row_v_cur, col_v_cur = cur
            rows_nxt, sg_nxt, ss_nxt, row_v_nxt, col_v_nxt = nxt

            @pl.when(jnp.logical_and(g % ch == 0, g + ch < nb))
            def _():
                qn = g // ch + 1

                @pl.when(qn % 2 == 0)
                def _():
                    pltpu.async_copy(packed_hbm.at[wid, qn],
                                     packed_cc.at[pl.ds(0, ch)], smeta)
                    pltpu.async_copy(w_hbm.at[wid, qn],
                                     w_cc.at[pl.ds(0, ch)], smeta)

                @pl.when(qn % 2 == 1)
                def _():
                    pltpu.async_copy(packed_hbm.at[wid, qn],
                                     packed_cc.at[pl.ds(ch, ch)], smeta)
                    pltpu.async_copy(w_hbm.at[wid, qn],
                                     w_cc.at[pl.ds(ch, ch)], smeta)

            @pl.when(g >= 2)
            def _():
                pltpu.make_async_copy(rows_nxt, acc.at[col_v_nxt],
                                      ss_nxt).wait()

            @pl.when(g + 1 < nb)
            def _():
                unpack(g + 1, row_v_nxt, col_v_nxt)
                pltpu.async_copy(tbl_hbm.at[row_v_nxt], rows_nxt, sg_nxt)

            pltpu.make_async_copy(tbl_hbm.at[row_v_cur], rows_cur,
                                  sg_cur).wait()

            ridx = ((g // ch) % 2) * ch + g % ch
            for j in range(EB // LANES):
                sl = pl.ds(j * LANES, LANES)
                w16 = w_cc[ridx, sl]
                for l in range(LANES):
                    wb = jnp.full((LANES,), w16[l], jnp.float32)
                    ee = j * LANES + l
                    for kk in range(d // LANES):
                        fsl = pl.ds(kk * LANES, LANES)
                        rows_cur[ee, fsl] = rows_cur[ee, fsl] * wb

            pltpu.async_copy(rows_cur, acc.at[col_v_cur], ss_cur, add=True)

        b0 = (rows_0, sg0, ss0, row_v0, col_v0)
        b1 = (rows_1, sg1, ss1, row_v1, col_v1)
        b2 = (rows_2, sg2, ss2, row_v2, col_v2)

        unpack(0, row_v0, col_v0)
        pltpu.async_copy(tbl_hbm.at[row_v0], rows_0, sg0)

        def body(i, _):
            g = i * 3
            phase(g, b0, b1)

            @pl.when(g + 1 < nb)
            def _():
                phase(g + 1, b1, b2)

            @pl.when(g + 2 < nb)
            def _():
                phase(g + 2, b2, b0)

            return 0

        lax.fori_loop(0, (nb + 2) // 3, body, 0)
        last = [b0, b1, b2][(nb - 2) % 3]
        pltpu.make_async_copy(last[0], acc.at[last[4]], last[2]).wait()
        last = [b0, b1, b2][(nb - 1) % 3]
        pltpu.make_async_copy(last[0], acc.at[last[4]], last[2]).wait()

    return k


def _tc_matmul1(x, w1, bm):
    m, cin = x.shape
    ch = w1.shape[1]
    hd = ch // 2

    def body(x_ref, w_ref, h0_ref, h1_ref):
        h = jnp.dot(x_ref[...], w_ref[...], preferred_element_type=jnp.float32)
        h0_ref[...] = h[:, :hd]
        h1_ref[...] = h[:, hd:]

    return pl.pallas_call(
        body,
        grid=(m // bm,),
        in_specs=[
            pl.BlockSpec((bm, cin), lambda i: (i, 0)),
            pl.BlockSpec((cin, ch), lambda i: (0, 0)),
        ],
        out_specs=[
            pl.BlockSpec((bm, hd), lambda i: (i, 0)),
            pl.BlockSpec((bm, hd), lambda i: (i, 0)),
        ],
        out_shape=[jax.ShapeDtypeStruct((m, hd), jnp.float32)] * 2,
    )(x, w1)


def _tc_mid(p00, p01, p10, p11, h10, h11, dis2, b10, b11, w2a, w2b, bm):
    m, hd = h10.shape
    cout = w2a.shape[1]

    def body(p00_r, p01_r, p10_r, p11_r, h10_r, h11_r, d_r, b10_r, b11_r,
             w2a_r, w2b_r, h2_ref):
        dcol = d_r[...]
        d2 = dcol * dcol
        a0 = jnp.maximum(dcol * (p00_r[...] + p01_r[...]) + d2 * h10_r[...]
                         + b10_r[...], 0.0)
        a1 = jnp.maximum(dcol * (p10_r[...] + p11_r[...]) + d2 * h11_r[...]
                         + b11_r[...], 0.0)
        h2_ref[...] = (jnp.dot(a0, w2a_r[...], preferred_element_type=jnp.float32)
                       + jnp.dot(a1, w2b_r[...], preferred_element_type=jnp.float32))

    mspec = pl.BlockSpec((bm, hd), lambda i: (i, 0))
    return pl.pallas_call(
        body,
        grid=(m // bm,),
        in_specs=[
            mspec, mspec, mspec, mspec, mspec, mspec,
            pl.BlockSpec((bm, 1), lambda i: (i, 0)),
            pl.BlockSpec((1, hd), lambda i: (0, 0)),
            pl.BlockSpec((1, hd), lambda i: (0, 0)),
            pl.BlockSpec((hd, cout), lambda i: (0, 0)),
            pl.BlockSpec((hd, cout), lambda i: (0, 0)),
        ],
        out_specs=pl.BlockSpec((bm, cout), lambda i: (i, 0)),
        out_shape=jax.ShapeDtypeStruct((m, cout), jnp.float32),
    )(p00, p01, p10, p11, h10, h11, dis2, b10, b11, w2a, w2b)


def _tc_final(q0, q1, h2, dis2, b2r, bm):
    m, cout = h2.shape

    def body(q0_r, q1_r, h2_r, d_r, b_r, out_ref):
        dcol = d_r[...]
        out_ref[...] = (dcol * (q0_r[...] + q1_r[...])
                        + dcol * dcol * h2_r[...] + b_r[...])

    mspec = pl.BlockSpec((bm, cout), lambda i: (i, 0))
    return pl.pallas_call(
        body,
        grid=(m // bm,),
        in_specs=[
            mspec, mspec, mspec,
            pl.BlockSpec((bm, 1), lambda i: (i, 0)),
            pl.BlockSpec((1, cout), lambda i: (0, 0)),
        ],
        out_specs=mspec,
        out_shape=jax.ShapeDtypeStruct((m, cout), jnp.float32),
    )(q0, q1, h2, dis2, b2r)


def kernel(x, edge_index, edge_weight, W1, b1, W2, b2):
    n, cin = x.shape
    e = edge_index.shape[1]
    ch = W1.shape[1]
    cout = W2.shape[1]
    hd = ch // 2
    npad = ((n + NS * LANES - 1) // (NS * LANES)) * (NS * LANES)
    bm = 1000
    nw = NC * NS
    nb = e // (nw * EB)

    row = edge_index[0].reshape(nw, nb, EB)
    col = edge_index[1].reshape(nw, nb, EB)
    ew = edge_weight.reshape(nw, nb, EB)
    packed = (col << 16) | row

    chq = 25
    nq = nb // chq
    packed4 = packed.reshape(nw, nq, chq, EB)

    h10, h11 = _tc_matmul1(x, W1, bm)
    deg_p = _make_deg_kernel(e, npad)(col, ew)
    dis_full = _tc_dis(deg_p.reshape(2 * npad // 128, 128)).reshape(npad)
    w4 = _make_wgen_kernel(e, npad)(row, ew, dis_full).reshape(nw, nq, chq, EB)
    agg_h = _make_agg_kernel(e, npad, hd)
    acc0 = agg_h(h10, packed4, w4)
    acc1 = agg_h(h11, packed4, w4)

    dis2 = dis_full[:n].reshape(n, 1)
    h2 = _tc_mid(acc0[0, :n], acc0[1, :n], acc1[0, :n], acc1[1, :n],
                 h10, h11, dis2,
                 b1[:hd].reshape(1, hd), b1[hd:].reshape(1, hd),
                 W2[:hd], W2[hd:], bm)

    acc2 = _make_agg_kernel(e, npad, cout)(h2, packed4, w4)
    out = _tc_final(acc2[0, :n], acc2[1, :n], h2, dis2, b2.reshape(1, cout), bm)
    return out
